# Initial kernel scaffold; baseline (speedup 1.0000x reference)
import jax
import jax.numpy as jnp
from jax.experimental import pallas as pl


def kernel(x, edge_index, W1, b1, W2, b2):
    raise NotImplementedError("write your pallas kernel here")


# speedup vs baseline: 21.6045x; 21.6045x over previous
import functools

import jax
import jax.numpy as jnp
from jax import lax
from jax.experimental import pallas as pl
from jax.experimental.pallas import tpu as pltpu
from jax.experimental.pallas import tpu_sc as plsc

N_NODES = 10000
N_EDGES = 320000
C_IN = 128
C_HID = 128
C_OUT = 64

NC = 2
NS = 16
NW = NC * NS
PAD_N = 10240
PAD_E = NW * 10240
E_PER_TILE = PAD_E // NW
CHUNK = 128
CHUNKS = E_PER_TILE // CHUNK
R_PER_TILE = PAD_N // NS
PAD_ROT = PAD_N - N_NODES

def _z16():
    return jnp.zeros((16,), jnp.float32)


def _o16():
    return jnp.ones((16,), jnp.float32)


def _sc_degree(dst_pad):
    mesh = plsc.VectorSubcoreMesh(core_axis_name="c", subcore_axis_name="s")

    @functools.partial(
        pl.kernel,
        out_type=jax.ShapeDtypeStruct((NC, NS, R_PER_TILE), jnp.float32),
        mesh=mesh,
        scratch_types=[
            pltpu.VMEM((CHUNK,), jnp.int32),
            pltpu.VMEM((CHUNK,), jnp.float32),
            pltpu.VMEM((R_PER_TILE,), jnp.float32),
            pltpu.VMEM_SHARED((PAD_N,), jnp.float32),
        ],
    )
    def k(dst_hbm, out_hbm, dstbuf, onesbuf, stage, acc):
        c = lax.axis_index("c")
        s = lax.axis_index("s")
        wid = c * NS + s

        for j in range(8):
            onesbuf[pl.ds(j * 16, 16)] = _o16()

        def zfill(j, carry):
            stage[pl.ds(j * 16, 16)] = _z16()
            return carry
        lax.fori_loop(0, R_PER_TILE // 16, zfill, 0)
        pltpu.sync_copy(stage, acc.at[pl.ds(s * R_PER_TILE, R_PER_TILE)])
        plsc.subcore_barrier()

        def body(i, carry):
            base = wid * E_PER_TILE + i * CHUNK
            pltpu.sync_copy(dst_hbm.at[pl.ds(base, CHUNK)], dstbuf)
            pltpu.sync_copy(onesbuf, acc.at[dstbuf], add=True)
            return carry
        lax.fori_loop(0, CHUNKS, body, 0)
        plsc.subcore_barrier()

        pltpu.sync_copy(acc.at[pl.ds(s * R_PER_TILE, R_PER_TILE)], stage)
        pltpu.sync_copy(stage, out_hbm.at[c, s])

    return k(dst_pad)


def _sc_edge_sum(g_pad, src_pad, dst_pad):
    mesh = plsc.VectorSubcoreMesh(core_axis_name="c", subcore_axis_name="s")

    @functools.partial(
        pl.kernel,
        out_type=jax.ShapeDtypeStruct((NC, PAD_N, C_HID), jnp.float32),
        mesh=mesh,
        scratch_types=[
            pltpu.VMEM((CHUNK,), jnp.int32),
            pltpu.VMEM((CHUNK,), jnp.int32),
            pltpu.VMEM((CHUNK, C_HID), jnp.float32),
            pltpu.SemaphoreType.DMA,
            pltpu.VMEM_SHARED((PAD_N, C_HID), jnp.float32),
        ],
    )
    def k(g_hbm, src_hbm, dst_hbm, out_hbm, srcbuf, dstbuf, rows, sem, acc):
        c = lax.axis_index("c")
        s = lax.axis_index("s")
        wid = c * NS + s

        def zrow(i, carry):
            for j in range(C_HID // 16):
                rows[i, pl.ds(j * 16, 16)] = _z16()
            return carry
        lax.fori_loop(0, CHUNK, zrow, 0)
        for kk in range(R_PER_TILE // CHUNK):
            pltpu.sync_copy(
                rows, acc.at[pl.ds(s * R_PER_TILE + kk * CHUNK, CHUNK)])
        plsc.subcore_barrier()

        def body(i, carry):
            base = wid * E_PER_TILE + i * CHUNK
            pltpu.sync_copy(src_hbm.at[pl.ds(base, CHUNK)], srcbuf)
            pltpu.sync_copy(dst_hbm.at[pl.ds(base, CHUNK)], dstbuf)
            pltpu.async_copy(g_hbm.at[srcbuf], rows, sem).wait()
            pltpu.sync_copy(rows, acc.at[dstbuf], add=True)
            return carry
        lax.fori_loop(0, CHUNKS, body, 0)
        plsc.subcore_barrier()

        for kk in range(R_PER_TILE // CHUNK):
            r0 = s * R_PER_TILE + kk * CHUNK
            pltpu.sync_copy(acc.at[pl.ds(r0, CHUNK)], rows)
            pltpu.sync_copy(rows, out_hbm.at[c, pl.ds(r0, CHUNK)])

    return k(g_pad, src_pad, dst_pad)


def _tc_transform(x_pad, W1, degT):
    B = 2048
    grid = (PAD_N // B,)

    def body(x_ref, w_ref, d_ref, g_ref, dinv_ref):
        d2 = d_ref[...]
        dinv = lax.rsqrt(d2[:, 0:1] + d2[:, 1:2] + 1.0)
        h = jnp.dot(x_ref[...], w_ref[...], preferred_element_type=jnp.float32)
        g_ref[...] = h * dinv
        dinv_ref[...] = dinv

    return pl.pallas_call(
        body,
        grid=grid,
        in_specs=[
            pl.BlockSpec((B, C_IN), lambda i: (i, 0)),
            pl.BlockSpec((C_IN, C_HID), lambda i: (0, 0)),
            pl.BlockSpec((B, 2), lambda i: (i, 0)),
        ],
        out_specs=[
            pl.BlockSpec((B, C_HID), lambda i: (i, 0)),
            pl.BlockSpec((B, 1), lambda i: (i, 0)),
        ],
        out_shape=[
            jax.ShapeDtypeStruct((PAD_N, C_HID), jnp.float32),
            jax.ShapeDtypeStruct((PAD_N, 1), jnp.float32),
        ],
    )(x_pad, W1, degT)


def _tc_head(s0, s1, g_pad, dinv, b1, W2, b2):
    B = 2048
    grid = (PAD_N // B,)

    def body(s0_ref, s1_ref, g_ref, di_ref, b1_ref, w2_ref, b2_ref, o_ref):
        z = di_ref[...] * (s0_ref[...] + s1_ref[...] + g_ref[...]) + b1_ref[...]
        z = jnp.maximum(z, 0.0)
        o_ref[...] = (
            jnp.dot(z, w2_ref[...], preferred_element_type=jnp.float32)
            + b2_ref[...]
        )

    return pl.pallas_call(
        body,
        grid=grid,
        in_specs=[
            pl.BlockSpec((B, C_HID), lambda i: (i, 0)),
            pl.BlockSpec((B, C_HID), lambda i: (i, 0)),
            pl.BlockSpec((B, C_HID), lambda i: (i, 0)),
            pl.BlockSpec((B, 1), lambda i: (i, 0)),
            pl.BlockSpec((1, C_HID), lambda i: (0, 0)),
            pl.BlockSpec((C_HID, C_OUT), lambda i: (0, 0)),
            pl.BlockSpec((1, C_OUT), lambda i: (0, 0)),
        ],
        out_specs=pl.BlockSpec((B, C_OUT), lambda i: (i, 0)),
        out_shape=jax.ShapeDtypeStruct((PAD_N, C_OUT), jnp.float32),
    )(s0, s1, g_pad, dinv, b1, W2, b2)


def kernel(x, edge_index, W1, b1, W2, b2):
    src = edge_index[0].astype(jnp.int32)
    dst = edge_index[1].astype(jnp.int32)

    pad_ids = (N_NODES + (jnp.arange(PAD_E - N_EDGES) % PAD_ROT)).astype(jnp.int32)
    src_pad = jnp.concatenate([src, pad_ids])
    dst_pad = jnp.concatenate([dst, pad_ids])
    x_pad = jnp.concatenate(
        [x, jnp.zeros((PAD_N - N_NODES, C_IN), jnp.float32)])

    deg = _sc_degree(dst_pad)
    degT = deg.reshape(NC, PAD_N).T
    g_pad, dinv = _tc_transform(x_pad, W1, degT)
    s_part = _sc_edge_sum(g_pad, src_pad, dst_pad)
    out = _tc_head(s_part[0], s_part[1], g_pad, dinv,
                   b1.reshape(1, C_HID), W2, b2.reshape(1, C_OUT))
    return out[:N_NODES]


# speedup vs baseline: 34.9280x; 1.6167x over previous
import functools

import jax
import jax.numpy as jnp
from jax import lax
from jax.experimental import pallas as pl
from jax.experimental.pallas import tpu as pltpu
from jax.experimental.pallas import tpu_sc as plsc

N_NODES = 10000
N_EDGES = 320000
C_IN = 128
C_HID = 128
C_OUT = 64

NC = 2
NS = 16
NW = NC * NS
PAD_N = 10240
PAD_E = NW * 10240
E_PER_TILE = PAD_E // NW
CHUNK = 128
CHUNKS = E_PER_TILE // CHUNK
R_PER_TILE = PAD_N // NS
PAD_ROT = PAD_N - N_NODES

def _z16():
    return jnp.zeros((16,), jnp.float32)


def _o16():
    return jnp.ones((16,), jnp.float32)


def _sc_degree(dst_pad):
    mesh = plsc.VectorSubcoreMesh(core_axis_name="c", subcore_axis_name="s")

    @functools.partial(
        pl.kernel,
        out_type=jax.ShapeDtypeStruct((NC, NS, R_PER_TILE), jnp.float32),
        mesh=mesh,
        scratch_types=[
            pltpu.VMEM((CHUNK,), jnp.int32),
            pltpu.VMEM((CHUNK,), jnp.float32),
            pltpu.VMEM((R_PER_TILE,), jnp.float32),
            pltpu.VMEM_SHARED((PAD_N,), jnp.float32),
        ],
    )
    def k(dst_hbm, out_hbm, dstbuf, onesbuf, stage, acc):
        c = lax.axis_index("c")
        s = lax.axis_index("s")
        wid = c * NS + s

        for j in range(8):
            onesbuf[pl.ds(j * 16, 16)] = _o16()

        def zfill(j, carry):
            stage[pl.ds(j * 16, 16)] = _z16()
            return carry
        lax.fori_loop(0, R_PER_TILE // 16, zfill, 0)
        pltpu.sync_copy(stage, acc.at[pl.ds(s * R_PER_TILE, R_PER_TILE)])
        plsc.subcore_barrier()

        def body(i, carry):
            base = wid * E_PER_TILE + i * CHUNK
            pltpu.sync_copy(dst_hbm.at[pl.ds(base, CHUNK)], dstbuf)
            pltpu.sync_copy(onesbuf, acc.at[dstbuf], add=True)
            return carry
        lax.fori_loop(0, CHUNKS, body, 0)
        plsc.subcore_barrier()

        pltpu.sync_copy(acc.at[pl.ds(s * R_PER_TILE, R_PER_TILE)], stage)
        pltpu.sync_copy(stage, out_hbm.at[c, s])

    return k(dst_pad)


BCH = 8
NBATCH = CHUNKS // BCH


def _sc_edge_sum(g_pad, idx5):
    mesh = plsc.VectorSubcoreMesh(core_axis_name="c", subcore_axis_name="s")

    @functools.partial(
        pl.kernel,
        out_type=jax.ShapeDtypeStruct((NC, PAD_N, C_HID), jnp.float32),
        mesh=mesh,
        scratch_types=[
            pltpu.VMEM((BCH, 2, CHUNK), jnp.int32),
            pltpu.VMEM((BCH, 2, CHUNK), jnp.int32),
            pltpu.VMEM((CHUNK, C_HID), jnp.float32),
            pltpu.VMEM((CHUNK, C_HID), jnp.float32),
            pltpu.SemaphoreType.DMA,
            pltpu.SemaphoreType.DMA,
            pltpu.SemaphoreType.DMA,
            pltpu.SemaphoreType.DMA,
            pltpu.SemaphoreType.DMA,
            pltpu.SemaphoreType.DMA,
            pltpu.VMEM_SHARED((PAD_N, C_HID), jnp.float32),
        ],
    )
    def k(g_hbm, idx_hbm, out_hbm, ib0, ib1,
          r0, r1, si0, si1, sg0, sg1, ss0, ss1, acc):
        c_ax = lax.axis_index("c")
        s_ax = lax.axis_index("s")
        wid = c_ax * NS + s_ax
        ib = (ib0, ib1)
        rows = (r0, r1)
        si = (si0, si1)
        sg = (sg0, sg1)
        ss = (ss0, ss1)

        def zrow(i, carry):
            for j in range(C_HID // 16):
                r0[i, pl.ds(j * 16, 16)] = _z16()
            return carry
        lax.fori_loop(0, CHUNK, zrow, 0)
        for kk in range(R_PER_TILE // CHUNK):
            pltpu.sync_copy(
                r0, acc.at[pl.ds(s_ax * R_PER_TILE + kk * CHUNK, CHUNK)])
        plsc.subcore_barrier()

        def startI(bn, slot):
            pltpu.async_copy(idx_hbm.at[wid, bn], ib[slot], si[slot])

        def waitI(slot):
            pltpu.make_async_copy(idx_hbm.at[wid, 0], ib[slot], si[slot]).wait()

        def startG(src_ref, b):
            pltpu.async_copy(g_hbm.at[src_ref], rows[b], sg[b])

        def waitG(src_ref, b):
            pltpu.make_async_copy(g_hbm.at[src_ref], rows[b], sg[b]).wait()

        def startS(dst_ref, b):
            pltpu.async_copy(rows[b], acc.at[dst_ref], ss[b], add=True)

        def waitS(dst_ref, b):
            pltpu.make_async_copy(rows[b], acc.at[dst_ref], ss[b]).wait()

        def step(kk, slot, pslot):
            b = kk % 2
            if kk < 2:
                waitS(ib[pslot].at[6 + kk, 1], b)
            else:
                waitS(ib[slot].at[kk - 2, 1], b)
            startG(ib[slot].at[kk, 0], b)
            if kk >= 1:
                waitG(ib[slot].at[kk - 1, 0], 1 - b)
                startS(ib[slot].at[kk - 1, 1], 1 - b)
            else:
                waitG(ib[pslot].at[7, 0], 1 - b)
                startS(ib[pslot].at[7, 1], 1 - b)

        pltpu.sync_copy(idx_hbm.at[wid, 0], ib0)
        startG(ib0.at[0, 0], 0)
        startG(ib0.at[1, 0], 1)
        waitG(ib0.at[0, 0], 0)
        startS(ib0.at[0, 1], 0)
        startI(1, 1)
        for kk in range(2, BCH):
            step(kk, 0, 0)

        def pair_body(p, carry):
            for (slot, pslot, boff) in ((1, 0, 1), (0, 1, 2)):
                waitI(slot)
                step(0, slot, pslot)
                step(1, slot, pslot)
                startI(2 * p + boff + 1, 1 - slot)
                for kk in range(2, BCH):
                    step(kk, slot, pslot)
            return carry
        lax.fori_loop(0, 4, pair_body, 0)

        waitI(1)
        for kk in range(BCH):
            step(kk, 1, 0)
        waitG(ib1.at[7, 0], 1)
        startS(ib1.at[7, 1], 1)
        waitS(ib1.at[6, 1], 0)
        waitS(ib1.at[7, 1], 1)
        plsc.subcore_barrier()

        for kk in range(R_PER_TILE // CHUNK):
            rr = s_ax * R_PER_TILE + kk * CHUNK
            pltpu.sync_copy(acc.at[pl.ds(rr, CHUNK)], r0)
            pltpu.sync_copy(r0, out_hbm.at[c_ax, pl.ds(rr, CHUNK)])

    return k(g_pad, idx5)


def _tc_transform(x_pad, W1, degT):
    B = 2048
    grid = (PAD_N // B,)

    def body(x_ref, w_ref, d_ref, g_ref, dinv_ref):
        d2 = d_ref[...]
        dinv = lax.rsqrt(d2[:, 0:1] + d2[:, 1:2] + 1.0)
        h = jnp.dot(x_ref[...], w_ref[...], preferred_element_type=jnp.float32)
        g_ref[...] = h * dinv
        dinv_ref[...] = dinv

    return pl.pallas_call(
        body,
        grid=grid,
        in_specs=[
            pl.BlockSpec((B, C_IN), lambda i: (i, 0)),
            pl.BlockSpec((C_IN, C_HID), lambda i: (0, 0)),
            pl.BlockSpec((B, 2), lambda i: (i, 0)),
        ],
        out_specs=[
            pl.BlockSpec((B, C_HID), lambda i: (i, 0)),
            pl.BlockSpec((B, 1), lambda i: (i, 0)),
        ],
        out_shape=[
            jax.ShapeDtypeStruct((PAD_N, C_HID), jnp.float32),
            jax.ShapeDtypeStruct((PAD_N, 1), jnp.float32),
        ],
    )(x_pad, W1, degT)


def _tc_head(s0, s1, g_pad, dinv, b1, W2, b2):
    B = 2048
    grid = (PAD_N // B,)

    def body(s0_ref, s1_ref, g_ref, di_ref, b1_ref, w2_ref, b2_ref, o_ref):
        z = di_ref[...] * (s0_ref[...] + s1_ref[...] + g_ref[...]) + b1_ref[...]
        z = jnp.maximum(z, 0.0)
        o_ref[...] = (
            jnp.dot(z, w2_ref[...], preferred_element_type=jnp.float32)
            + b2_ref[...]
        )

    return pl.pallas_call(
        body,
        grid=grid,
        in_specs=[
            pl.BlockSpec((B, C_HID), lambda i: (i, 0)),
            pl.BlockSpec((B, C_HID), lambda i: (i, 0)),
            pl.BlockSpec((B, C_HID), lambda i: (i, 0)),
            pl.BlockSpec((B, 1), lambda i: (i, 0)),
            pl.BlockSpec((1, C_HID), lambda i: (0, 0)),
            pl.BlockSpec((C_HID, C_OUT), lambda i: (0, 0)),
            pl.BlockSpec((1, C_OUT), lambda i: (0, 0)),
        ],
        out_specs=pl.BlockSpec((B, C_OUT), lambda i: (i, 0)),
        out_shape=jax.ShapeDtypeStruct((PAD_N, C_OUT), jnp.float32),
    )(s0, s1, g_pad, dinv, b1, W2, b2)


def kernel(x, edge_index, W1, b1, W2, b2):
    src = edge_index[0].astype(jnp.int32)
    dst = edge_index[1].astype(jnp.int32)

    pad_ids = (N_NODES + (jnp.arange(PAD_E - N_EDGES) % PAD_ROT)).astype(jnp.int32)
    src_pad = jnp.concatenate([src, pad_ids])
    dst_pad = jnp.concatenate([dst, pad_ids])
    x_pad = jnp.concatenate(
        [x, jnp.zeros((PAD_N - N_NODES, C_IN), jnp.float32)])

    src3 = src_pad.reshape(NW, CHUNKS, CHUNK)
    dst3 = dst_pad.reshape(NW, CHUNKS, CHUNK)
    idx5 = jnp.stack([src3, dst3], axis=2).reshape(NW, NBATCH, BCH, 2, CHUNK)

    deg = _sc_degree(dst_pad)
    degT = deg.reshape(NC, PAD_N).T
    g_pad, dinv = _tc_transform(x_pad, W1, degT)
    s_part = _sc_edge_sum(g_pad, idx5)
    out = _tc_head(s_part[0], s_part[1], g_pad, dinv,
                   b1.reshape(1, C_HID), W2, b2.reshape(1, C_OUT))
    return out[:N_NODES]


# speedup vs baseline: 41.6421x; 1.1922x over previous
import functools

import jax
import jax.numpy as jnp
from jax import lax
from jax.experimental import pallas as pl
from jax.experimental.pallas import tpu as pltpu
from jax.experimental.pallas import tpu_sc as plsc

N_NODES = 10000
N_EDGES = 320000
C_IN = 128
C_HID = 128
C_OUT = 64

NC = 2
NS = 16
NW = NC * NS
PAD_N = 10240
PAD_E = NW * 10240
E_PER_TILE = PAD_E // NW
CHUNK = 128
CHUNKS = E_PER_TILE // CHUNK
R_PER_TILE = PAD_N // NS
PAD_ROT = PAD_N - N_NODES

def _z16():
    return jnp.zeros((16,), jnp.float32)


def _o16():
    return jnp.ones((16,), jnp.float32)


def _sc_degree(idx5):
    mesh = plsc.VectorSubcoreMesh(core_axis_name="c", subcore_axis_name="s")

    @functools.partial(
        pl.kernel,
        out_type=jax.ShapeDtypeStruct((NC, NS, R_PER_TILE), jnp.float32),
        mesh=mesh,
        scratch_types=[
            pltpu.VMEM((BCH, 2, CHUNK), jnp.int32),
            pltpu.VMEM((BCH, 2, CHUNK), jnp.int32),
            pltpu.VMEM((CHUNK,), jnp.float32),
            pltpu.VMEM((R_PER_TILE,), jnp.float32),
            pltpu.SemaphoreType.DMA,
            pltpu.SemaphoreType.DMA,
            pltpu.SemaphoreType.DMA,
            pltpu.SemaphoreType.DMA,
            pltpu.VMEM_SHARED((PAD_N,), jnp.float32),
        ],
    )
    def k(idx_hbm, out_hbm, ib0, ib1, onesbuf, stage, si0, si1, ss0, ss1, acc):
        c = lax.axis_index("c")
        s = lax.axis_index("s")
        wid = c * NS + s
        ib = (ib0, ib1)
        si = (si0, si1)
        ss = (ss0, ss1)

        for j in range(CHUNK // 16):
            onesbuf[pl.ds(j * 16, 16)] = _o16()

        def zfill(j, carry):
            stage[pl.ds(j * 16, 16)] = _z16()
            return carry
        lax.fori_loop(0, R_PER_TILE // 16, zfill, 0)
        pltpu.sync_copy(stage, acc.at[pl.ds(s * R_PER_TILE, R_PER_TILE)])
        plsc.subcore_barrier()

        def startI(bn, slot):
            pltpu.async_copy(idx_hbm.at[wid, bn], ib[slot], si[slot])

        def waitI(slot):
            pltpu.make_async_copy(idx_hbm.at[wid, 0], ib[slot], si[slot]).wait()

        def fire(slot):
            for kk in range(BCH):
                pltpu.async_copy(onesbuf, acc.at[ib[slot].at[kk, 1]],
                                 ss[slot], add=True)

        def drain(slot):
            for kk in range(BCH):
                pltpu.make_async_copy(onesbuf, acc.at[ib[slot].at[kk, 1]],
                                      ss[slot]).wait()

        pltpu.sync_copy(idx_hbm.at[wid, 0], ib0)
        fire(0)
        startI(1, 1)

        def pair(p, carry):
            waitI(1)
            fire(1)
            drain(0)
            startI(2 * p + 2, 0)
            waitI(0)
            fire(0)
            drain(1)
            startI(2 * p + 3, 1)
            return carry
        lax.fori_loop(0, (NBATCH - 2) // 2, pair, 0)

        waitI(1)
        fire(1)
        drain(0)
        drain(1)
        plsc.subcore_barrier()

        pltpu.sync_copy(acc.at[pl.ds(s * R_PER_TILE, R_PER_TILE)],
                        out_hbm.at[c, s])

    return k(idx5)


BCH = 8
NBATCH = CHUNKS // BCH


def _sc_edge_sum(g_pad, idx5):
    mesh = plsc.VectorSubcoreMesh(core_axis_name="c", subcore_axis_name="s")

    @functools.partial(
        pl.kernel,
        out_type=jax.ShapeDtypeStruct((NC, PAD_N, C_HID), jnp.float32),
        mesh=mesh,
        scratch_types=[
            pltpu.VMEM((BCH, 2, CHUNK), jnp.int32),
            pltpu.VMEM((BCH, 2, CHUNK), jnp.int32),
            pltpu.VMEM((CHUNK, C_HID), jnp.float32),
            pltpu.VMEM((CHUNK, C_HID), jnp.float32),
            pltpu.SemaphoreType.DMA,
            pltpu.SemaphoreType.DMA,
            pltpu.SemaphoreType.DMA,
            pltpu.SemaphoreType.DMA,
            pltpu.SemaphoreType.DMA,
            pltpu.SemaphoreType.DMA,
            pltpu.VMEM_SHARED((PAD_N, C_HID), jnp.float32),
        ],
    )
    def k(g_hbm, idx_hbm, out_hbm, ib0, ib1,
          r0, r1, si0, si1, sg0, sg1, ss0, ss1, acc):
        c_ax = lax.axis_index("c")
        s_ax = lax.axis_index("s")
        wid = c_ax * NS + s_ax
        ib = (ib0, ib1)
        rows = (r0, r1)
        si = (si0, si1)
        sg = (sg0, sg1)
        ss = (ss0, ss1)

        def zrow(i, carry):
            for j in range(C_HID // 16):
                r0[i, pl.ds(j * 16, 16)] = _z16()
            return carry
        lax.fori_loop(0, CHUNK, zrow, 0)
        for kk in range(R_PER_TILE // CHUNK):
            pltpu.sync_copy(
                r0, acc.at[pl.ds(s_ax * R_PER_TILE + kk * CHUNK, CHUNK)])
        plsc.subcore_barrier()

        def startI(bn, slot):
            pltpu.async_copy(idx_hbm.at[wid, bn], ib[slot], si[slot])

        def waitI(slot):
            pltpu.make_async_copy(idx_hbm.at[wid, 0], ib[slot], si[slot]).wait()

        def startG(src_ref, b):
            pltpu.async_copy(g_hbm.at[src_ref], rows[b], sg[b])

        def waitG(src_ref, b):
            pltpu.make_async_copy(g_hbm.at[src_ref], rows[b], sg[b]).wait()

        def startS(dst_ref, b):
            pltpu.async_copy(rows[b], acc.at[dst_ref], ss[b], add=True)

        def waitS(dst_ref, b):
            pltpu.make_async_copy(rows[b], acc.at[dst_ref], ss[b]).wait()

        def step(kk, slot, pslot):
            b = kk % 2
            if kk < 2:
                waitS(ib[pslot].at[6 + kk, 1], b)
            else:
                waitS(ib[slot].at[kk - 2, 1], b)
            startG(ib[slot].at[kk, 0], b)
            if kk >= 1:
                waitG(ib[slot].at[kk - 1, 0], 1 - b)
                startS(ib[slot].at[kk - 1, 1], 1 - b)
            else:
                waitG(ib[pslot].at[7, 0], 1 - b)
                startS(ib[pslot].at[7, 1], 1 - b)

        pltpu.sync_copy(idx_hbm.at[wid, 0], ib0)
        startG(ib0.at[0, 0], 0)
        startG(ib0.at[1, 0], 1)
        waitG(ib0.at[0, 0], 0)
        startS(ib0.at[0, 1], 0)
        startI(1, 1)
        for kk in range(2, BCH):
            step(kk, 0, 0)

        def pair_body(p, carry):
            for (slot, pslot, boff) in ((1, 0, 1), (0, 1, 2)):
                waitI(slot)
                step(0, slot, pslot)
                step(1, slot, pslot)
                startI(2 * p + boff + 1, 1 - slot)
                for kk in range(2, BCH):
                    step(kk, slot, pslot)
            return carry
        lax.fori_loop(0, 4, pair_body, 0)

        waitI(1)
        for kk in range(BCH):
            step(kk, 1, 0)
        waitG(ib1.at[7, 0], 1)
        startS(ib1.at[7, 1], 1)
        waitS(ib1.at[6, 1], 0)
        waitS(ib1.at[7, 1], 1)
        plsc.subcore_barrier()

        rr = s_ax * R_PER_TILE
        pltpu.sync_copy(acc.at[pl.ds(rr, R_PER_TILE)],
                        out_hbm.at[c_ax, pl.ds(rr, R_PER_TILE)])

    return k(g_pad, idx5)


def _tc_transform(x_pad, W1, degT):
    B = 2048
    grid = (PAD_N // B,)

    def body(x_ref, w_ref, d_ref, g_ref, dinv_ref):
        d2 = d_ref[...]
        dinv = lax.rsqrt(d2[:, 0:1] + d2[:, 1:2] + 1.0)
        h = jnp.dot(x_ref[...], w_ref[...], preferred_element_type=jnp.float32)
        g_ref[...] = h * dinv
        dinv_ref[...] = dinv

    return pl.pallas_call(
        body,
        grid=grid,
        in_specs=[
            pl.BlockSpec((B, C_IN), lambda i: (i, 0)),
            pl.BlockSpec((C_IN, C_HID), lambda i: (0, 0)),
            pl.BlockSpec((B, 2), lambda i: (i, 0)),
        ],
        out_specs=[
            pl.BlockSpec((B, C_HID), lambda i: (i, 0)),
            pl.BlockSpec((B, 1), lambda i: (i, 0)),
        ],
        out_shape=[
            jax.ShapeDtypeStruct((PAD_N, C_HID), jnp.float32),
            jax.ShapeDtypeStruct((PAD_N, 1), jnp.float32),
        ],
    )(x_pad, W1, degT)


def _tc_head(s0, s1, g_pad, dinv, b1, W2, b2):
    B = 2048
    grid = (PAD_N // B,)

    def body(s0_ref, s1_ref, g_ref, di_ref, b1_ref, w2_ref, b2_ref, o_ref):
        z = di_ref[...] * (s0_ref[...] + s1_ref[...] + g_ref[...]) + b1_ref[...]
        z = jnp.maximum(z, 0.0)
        o_ref[...] = (
            jnp.dot(z, w2_ref[...], preferred_element_type=jnp.float32)
            + b2_ref[...]
        )

    return pl.pallas_call(
        body,
        grid=grid,
        in_specs=[
            pl.BlockSpec((B, C_HID), lambda i: (i, 0)),
            pl.BlockSpec((B, C_HID), lambda i: (i, 0)),
            pl.BlockSpec((B, C_HID), lambda i: (i, 0)),
            pl.BlockSpec((B, 1), lambda i: (i, 0)),
            pl.BlockSpec((1, C_HID), lambda i: (0, 0)),
            pl.BlockSpec((C_HID, C_OUT), lambda i: (0, 0)),
            pl.BlockSpec((1, C_OUT), lambda i: (0, 0)),
        ],
        out_specs=pl.BlockSpec((B, C_OUT), lambda i: (i, 0)),
        out_shape=jax.ShapeDtypeStruct((PAD_N, C_OUT), jnp.float32),
    )(s0, s1, g_pad, dinv, b1, W2, b2)


def kernel(x, edge_index, W1, b1, W2, b2):
    src = edge_index[0].astype(jnp.int32)
    dst = edge_index[1].astype(jnp.int32)

    pad_ids = (N_NODES + (jnp.arange(PAD_E - N_EDGES) % PAD_ROT)).astype(jnp.int32)
    src_pad = jnp.concatenate([src, pad_ids])
    dst_pad = jnp.concatenate([dst, pad_ids])
    x_pad = jnp.concatenate(
        [x, jnp.zeros((PAD_N - N_NODES, C_IN), jnp.float32)])

    src3 = src_pad.reshape(NW, CHUNKS, CHUNK)
    dst3 = dst_pad.reshape(NW, CHUNKS, CHUNK)
    idx5 = jnp.stack([src3, dst3], axis=2).reshape(NW, NBATCH, BCH, 2, CHUNK)

    deg = _sc_degree(idx5)
    degT = deg.reshape(NC, PAD_N).T
    g_pad, dinv = _tc_transform(x_pad, W1, degT)
    s_part = _sc_edge_sum(g_pad, idx5)
    out = _tc_head(s_part[0], s_part[1], g_pad, dinv,
                   b1.reshape(1, C_HID), W2, b2.reshape(1, C_OUT))
    return out[:N_NODES]


# speedup vs baseline: 44.1716x; 1.0607x over previous
import functools

import jax
import jax.numpy as jnp
from jax import lax
from jax.experimental import pallas as pl
from jax.experimental.pallas import tpu as pltpu
from jax.experimental.pallas import tpu_sc as plsc

N_NODES = 10000
N_EDGES = 320000
C_IN = 128
C_HID = 128
C_OUT = 64

NC = 2
NS = 16
NW = NC * NS
PAD_N = 10240
PAD_E = NW * 10240
E_PER_TILE = PAD_E // NW
CHUNK = 128
CHUNKS = E_PER_TILE // CHUNK
R_PER_TILE = PAD_N // NS
PAD_ROT = PAD_N - N_NODES
BCH = 8
NBATCH = CHUNKS // BCH
TB = 2000


def _z16():
    return jnp.zeros((16,), jnp.float32)


def _o16():
    return jnp.ones((16,), jnp.float32)


def _sc_degree(dstv):
    mesh = plsc.VectorSubcoreMesh(core_axis_name="c", subcore_axis_name="s")

    @functools.partial(
        pl.kernel,
        out_type=jax.ShapeDtypeStruct((NC, NS, R_PER_TILE), jnp.float32),
        mesh=mesh,
        scratch_types=[
            pltpu.VMEM((BCH, CHUNK), jnp.int32),
            pltpu.VMEM((BCH, CHUNK), jnp.int32),
            pltpu.VMEM((CHUNK,), jnp.float32),
            pltpu.VMEM((R_PER_TILE,), jnp.float32),
            pltpu.SemaphoreType.DMA,
            pltpu.SemaphoreType.DMA,
            pltpu.SemaphoreType.DMA,
            pltpu.SemaphoreType.DMA,
            pltpu.VMEM_SHARED((PAD_N,), jnp.float32),
        ],
    )
    def k(idx_hbm, out_hbm, ib0, ib1, onesbuf, stage, si0, si1, ss0, ss1, acc):
        c = lax.axis_index("c")
        s = lax.axis_index("s")
        wid = c * NS + s
        ib = (ib0, ib1)
        si = (si0, si1)
        ss = (ss0, ss1)

        for j in range(CHUNK // 16):
            onesbuf[pl.ds(j * 16, 16)] = _o16()

        def zfill(j, carry):
            stage[pl.ds(j * 16, 16)] = _z16()
            return carry
        lax.fori_loop(0, R_PER_TILE // 16, zfill, 0)
        pltpu.sync_copy(stage, acc.at[pl.ds(s * R_PER_TILE, R_PER_TILE)])
        plsc.subcore_barrier()

        def startI(bn, slot):
            pltpu.async_copy(idx_hbm.at[wid, bn], ib[slot], si[slot])

        def waitI(slot):
            pltpu.make_async_copy(idx_hbm.at[wid, 0], ib[slot], si[slot]).wait()

        def fire(slot):
            for kk in range(BCH):
                pltpu.async_copy(onesbuf, acc.at[ib[slot].at[kk]],
                                 ss[slot], add=True)

        def drain(slot):
            for kk in range(BCH):
                pltpu.make_async_copy(onesbuf, acc.at[ib[slot].at[kk]],
                                      ss[slot]).wait()

        pltpu.sync_copy(idx_hbm.at[wid, 0], ib0)
        fire(0)
        startI(1, 1)

        def pair(p, carry):
            waitI(1)
            fire(1)
            drain(0)
            startI(2 * p + 2, 0)
            waitI(0)
            fire(0)
            drain(1)
            startI(2 * p + 3, 1)
            return carry
        lax.fori_loop(0, (NBATCH - 2) // 2, pair, 0)

        waitI(1)
        fire(1)
        drain(0)
        drain(1)
        plsc.subcore_barrier()

        pltpu.sync_copy(acc.at[pl.ds(s * R_PER_TILE, R_PER_TILE)],
                        out_hbm.at[c, s])

    return k(dstv)


def _sc_edge_sum(g, srcv, dstv):
    mesh = plsc.VectorSubcoreMesh(core_axis_name="c", subcore_axis_name="s")

    @functools.partial(
        pl.kernel,
        out_type=jax.ShapeDtypeStruct((NC, PAD_N, C_HID), jnp.float32),
        mesh=mesh,
        scratch_types=[
            pltpu.VMEM((BCH, CHUNK), jnp.int32),
            pltpu.VMEM((BCH, CHUNK), jnp.int32),
            pltpu.VMEM((BCH, CHUNK), jnp.int32),
            pltpu.VMEM((BCH, CHUNK), jnp.int32),
            pltpu.VMEM((CHUNK, C_HID), jnp.float32),
            pltpu.VMEM((CHUNK, C_HID), jnp.float32),
            pltpu.SemaphoreType.DMA,
            pltpu.SemaphoreType.DMA,
            pltpu.SemaphoreType.DMA,
            pltpu.SemaphoreType.DMA,
            pltpu.SemaphoreType.DMA,
            pltpu.SemaphoreType.DMA,
            pltpu.VMEM_SHARED((PAD_N, C_HID), jnp.float32),
        ],
    )
    def k(g_hbm, src_hbm, dst_hbm, out_hbm, is0, is1, id0, id1,
          r0, r1, si0, si1, sg0, sg1, ss0, ss1, acc):
        c_ax = lax.axis_index("c")
        s_ax = lax.axis_index("s")
        wid = c_ax * NS + s_ax
        ibs = (is0, is1)
        ibd = (id0, id1)
        rows = (r0, r1)
        si = (si0, si1)
        sg = (sg0, sg1)
        ss = (ss0, ss1)

        def zrow(i, carry):
            for j in range(C_HID // 16):
                r0[i, pl.ds(j * 16, 16)] = _z16()
            return carry
        lax.fori_loop(0, CHUNK, zrow, 0)
        for kk in range(R_PER_TILE // CHUNK):
            pltpu.sync_copy(
                r0, acc.at[pl.ds(s_ax * R_PER_TILE + kk * CHUNK, CHUNK)])
        plsc.subcore_barrier()

        def startI(bn, slot):
            pltpu.async_copy(src_hbm.at[wid, bn], ibs[slot], si[slot])
            pltpu.async_copy(dst_hbm.at[wid, bn], ibd[slot], si[slot])

        def waitI(slot):
            pltpu.make_async_copy(src_hbm.at[wid, 0], ibs[slot], si[slot]).wait()
            pltpu.make_async_copy(dst_hbm.at[wid, 0], ibd[slot], si[slot]).wait()

        def startG(src_ref, b):
            pltpu.async_copy(g_hbm.at[src_ref], rows[b], sg[b])

        def waitG(src_ref, b):
            pltpu.make_async_copy(g_hbm.at[src_ref], rows[b], sg[b]).wait()

        def startS(dst_ref, b):
            pltpu.async_copy(rows[b], acc.at[dst_ref], ss[b], add=True)

        def waitS(dst_ref, b):
            pltpu.make_async_copy(rows[b], acc.at[dst_ref], ss[b]).wait()

        def step(kk, slot, pslot):
            b = kk % 2
            if kk < 2:
                waitS(ibd[pslot].at[6 + kk], b)
            else:
                waitS(ibd[slot].at[kk - 2], b)
            startG(ibs[slot].at[kk], b)
            if kk >= 1:
                waitG(ibs[slot].at[kk - 1], 1 - b)
                startS(ibd[slot].at[kk - 1], 1 - b)
            else:
                waitG(ibs[pslot].at[7], 1 - b)
                startS(ibd[pslot].at[7], 1 - b)

        pltpu.sync_copy(src_hbm.at[wid, 0], is0)
        pltpu.sync_copy(dst_hbm.at[wid, 0], id0)
        startG(is0.at[0], 0)
        startG(is0.at[1], 1)
        waitG(is0.at[0], 0)
        startS(id0.at[0], 0)
        startI(1, 1)
        for kk in range(2, BCH):
            step(kk, 0, 0)

        def pair_body(p, carry):
            for (slot, pslot, boff) in ((1, 0, 1), (0, 1, 2)):
                waitI(slot)
                step(0, slot, pslot)
                step(1, slot, pslot)
                startI(2 * p + boff + 1, 1 - slot)
                for kk in range(2, BCH):
                    step(kk, slot, pslot)
            return carry
        lax.fori_loop(0, 4, pair_body, 0)

        waitI(1)
        for kk in range(BCH):
            step(kk, 1, 0)
        waitG(is1.at[7], 1)
        startS(id1.at[7], 1)
        waitS(id1.at[6], 0)
        waitS(id1.at[7], 1)
        plsc.subcore_barrier()

        rr = s_ax * R_PER_TILE
        pltpu.sync_copy(acc.at[pl.ds(rr, R_PER_TILE)],
                        out_hbm.at[c_ax, pl.ds(rr, R_PER_TILE)])

    return k(g, srcv, dstv)


def _tc_transform(x, W1, degT):
    grid = (N_NODES // TB,)

    def body(x_ref, w_ref, d_ref, g_ref, dinv_ref):
        d2 = d_ref[...]
        dinv = lax.rsqrt(d2[:, 0:1] + d2[:, 1:2] + 1.0)
        h = jnp.dot(x_ref[...], w_ref[...], preferred_element_type=jnp.float32)
        g_ref[...] = h * dinv
        dinv_ref[...] = dinv

    return pl.pallas_call(
        body,
        grid=grid,
        in_specs=[
            pl.BlockSpec((TB, C_IN), lambda i: (i, 0)),
            pl.BlockSpec((C_IN, C_HID), lambda i: (0, 0)),
            pl.BlockSpec((TB, 2), lambda i: (i, 0)),
        ],
        out_specs=[
            pl.BlockSpec((TB, C_HID), lambda i: (i, 0)),
            pl.BlockSpec((TB, 1), lambda i: (i, 0)),
        ],
        out_shape=[
            jax.ShapeDtypeStruct((N_NODES, C_HID), jnp.float32),
            jax.ShapeDtypeStruct((N_NODES, 1), jnp.float32),
        ],
    )(x, W1, degT)


def _tc_head(s_part, g, dinv, b1, W2, b2):
    grid = (N_NODES // TB,)

    def body(s_ref, g_ref, di_ref, b1_ref, w2_ref, b2_ref, o_ref):
        z = (di_ref[...] * (s_ref[0] + s_ref[1] + g_ref[...])
             + b1_ref[...])
        z = jnp.maximum(z, 0.0)
        o_ref[...] = (
            jnp.dot(z, w2_ref[...], preferred_element_type=jnp.float32)
            + b2_ref[...]
        )

    return pl.pallas_call(
        body,
        grid=grid,
        in_specs=[
            pl.BlockSpec((NC, TB, C_HID), lambda i: (0, i, 0)),
            pl.BlockSpec((TB, C_HID), lambda i: (i, 0)),
            pl.BlockSpec((TB, 1), lambda i: (i, 0)),
            pl.BlockSpec((1, C_HID), lambda i: (0, 0)),
            pl.BlockSpec((C_HID, C_OUT), lambda i: (0, 0)),
            pl.BlockSpec((1, C_OUT), lambda i: (0, 0)),
        ],
        out_specs=pl.BlockSpec((TB, C_OUT), lambda i: (i, 0)),
        out_shape=jax.ShapeDtypeStruct((N_NODES, C_OUT), jnp.float32),
    )(s_part, g, dinv, b1, W2, b2)


def kernel(x, edge_index, W1, b1, W2, b2):
    src = edge_index[0].astype(jnp.int32)
    dst = edge_index[1].astype(jnp.int32)

    npad = PAD_E - N_EDGES
    pad_src = (jnp.arange(npad) % N_NODES).astype(jnp.int32)
    pad_dst = (N_NODES + (jnp.arange(npad) % PAD_ROT)).astype(jnp.int32)
    srcv = jnp.concatenate([src, pad_src]).reshape(NW, NBATCH, BCH, CHUNK)
    dstv = jnp.concatenate([dst, pad_dst]).reshape(NW, NBATCH, BCH, CHUNK)

    deg = _sc_degree(dstv)
    degT = deg.reshape(NC, PAD_N).T[:N_NODES]
    g, dinv = _tc_transform(x, W1, degT)
    s_part = _sc_edge_sum(g, srcv, dstv)
    return _tc_head(s_part, g, dinv,
                    b1.reshape(1, C_HID), W2, b2.reshape(1, C_OUT))
